# baseline (speedup 1.0000x reference)
import functools

import jax
import jax.numpy as jnp
from jax import lax
from jax.experimental import pallas as pl
from jax.experimental.pallas import tpu as pltpu
from jax.experimental.pallas import tpu_sc as plsc

N_NODES = 10000
N_EDGES = 320000
D = 128

NC = 2
NS = 16
NW = NC * NS

CHUNK = 64
E_PAD = 327680
N_CHUNK_ROWS = E_PAD // CHUNK
CH_T = N_CHUNK_ROWS // NS
CH_Q = CH_T // 4

HB = 10240
HB_ROWS = HB // D
AGG_ROWS = 10240
SHARE = AGG_ROWS // NS
DH = 64

RB = 400
GRID = N_NODES // RB

_mesh = plsc.VectorSubcoreMesh(core_axis_name="c", subcore_axis_name="s")

_f32 = jnp.float32
_i32 = jnp.int32


def _fused_body(dst2d_hbm, src2d_hbm, y0_hbm, y1_hbm, x0_hbm, x1_hbm,
                coef_hbm, out_hbm,
                dstv, sidx, hist1d, packb, rowids, histv, dinvv, dinv_lv,
                dumidx, coefv,
                rows_a, rows_b, rows_c, rows_d, sem_a, sem_b, sem_c, sem_d,
                sem_e, sem_f, sem_g, sem_h, hist_sh, ys_sp, agg):
    c = lax.axis_index("c")
    s = lax.axis_index("s")

    zeros16 = jnp.zeros((16,), _f32)
    ones16 = jnp.ones((16,), _f32)

    def _z1(i, carry):
        hist1d[pl.ds(i * 16, 16)] = zeros16
        return carry

    lax.fori_loop(0, HB // 16, _z1, 0)

    def _zp(r, carry):
        for cb in range(8):
            packb[r, pl.ds(cb * 16, 16)] = zeros16
        return carry

    lax.fori_loop(0, HB_ROWS // 4, _zp, 0)

    def _za(r, carry):
        for cb in range(DH // 16):
            rows_a[r, pl.ds(cb * 16, 16)] = zeros16
        return carry

    lax.fori_loop(0, CHUNK, _za, 0)

    for p in range(4):
        rowids[p, pl.ds(0, 16)] = lax.iota(_i32, 16) + p * 20
        rowids[p, pl.ds(4, 16)] = lax.iota(_i32, 16) + (p * 20 + 4)

    for k in range(SHARE // CHUNK):
        pltpu.sync_copy(rows_a, agg.at[pl.ds(s * SHARE + k * CHUNK, CHUNK)])

    @pl.when(s == 0)
    def _():
        for p in range(4):
            pltpu.sync_copy(packb, hist_sh.at[pl.ds(p * 20, 20)])

    plsc.subcore_barrier()

    def _hrow(r, carry):
        for cb in range(CHUNK // 16):
            idx = dstv[r, pl.ds(cb * 16, 16)]
            plsc.addupdate_scatter(hist1d, [idx], ones16)
        return carry

    for h in range(4):
        pltpu.sync_copy(dst2d_hbm.at[pl.ds(s * CH_T + h * CH_Q, CH_Q)], dstv)
        lax.fori_loop(0, CH_Q, _hrow, 0)

    for p in range(4):
        def _pk(r, carry, p=p):
            for cb in range(8):
                packb[r, pl.ds(cb * 16, 16)] = \
                    hist1d[pl.ds((p * 20 + r) * D + cb * 16, 16)]
            return carry

        lax.fori_loop(0, HB_ROWS // 4, _pk, 0)
        pltpu.sync_copy(packb, hist_sh.at[rowids.at[p]], add=True)

    plsc.subcore_barrier()

    pltpu.sync_copy(hist_sh.at[pl.ds(s * 5, 5)], histv)
    half16 = jnp.full((16,), 0.5, _f32)
    t32 = jnp.full((16,), 1.5, _f32)
    magic = jnp.full((16,), 0x5F3759DF, _i32)
    for r in range(5):
        for cb in range(8):
            x = histv[r, pl.ds(cb * 16, 16)]
            yv = plsc.bitcast(magic - (plsc.bitcast(x, _i32) >> 1), _f32)
            for _ in range(3):
                yv = yv * (t32 - half16 * x * yv * yv)
            dv = jnp.where(x > 0, yv, 0.0)
            dinvv[r, pl.ds(cb * 16, 16)] = dv
            dinv_lv[pl.ds((r * 8 + cb) * 16, 16)] = dv

    def _scale(y_hbm):
        def _q(q, carry):
            base = s * SHARE + q * CHUNK

            pltpu.sync_copy(y_hbm.at[pl.ds(base, CHUNK)], rows_a)

            def _r(r, carry2):
                gi = jnp.full((16,), q * CHUNK + r, _i32)
                dv = plsc.load_gather(dinv_lv, [gi])
                for cb in range(DH // 16):
                    rows_a[r, pl.ds(cb * 16, 16)] = rows_a[r, pl.ds(cb * 16, 16)] * dv
                return carry2

            lax.fori_loop(0, CHUNK, _r, 0)
            pltpu.sync_copy(rows_a, ys_sp.at[pl.ds(base, CHUNK)])
            return carry

        lax.fori_loop(0, SHARE // CHUNK, _q, 0)

    @pl.when(c == 0)
    def _():
        _scale(y0_hbm)

    @pl.when(c == 1)
    def _():
        _scale(y1_hbm)

    plsc.subcore_barrier()

    bufs = (rows_a, rows_b, rows_c, rows_d)
    gsems = (sem_a, sem_b, sem_c, sem_d)
    ssems = (sem_e, sem_f, sem_g, sem_h)

    for g in range(4):
        dumidx[pl.ds(g * 16, 16)] = lax.iota(_i32, 16) + (N_NODES + g * 16)
    for i in range(4):
        pltpu.async_copy(bufs[i], agg.at[dumidx], ssems[i], add=True)

    def _quad(k, carry):
        j0 = 4 * k
        for i in range(4):
            pltpu.make_async_copy(y0_hbm.at[pl.ds(0, CHUNK)], bufs[i],
                                  ssems[i]).wait()
        gs = [pltpu.async_copy(ys_sp.at[sidx.at[j0 + i]], bufs[i], gsems[i])
              for i in range(4)]
        for i in range(4):
            gs[i].wait()
            pltpu.async_copy(bufs[i], agg.at[dstv.at[j0 + i]], ssems[i],
                             add=True)
        return carry

    for h in range(4):
        pltpu.sync_copy(src2d_hbm.at[pl.ds(s * CH_T + h * CH_Q, CH_Q)], sidx)
        pltpu.sync_copy(dst2d_hbm.at[pl.ds(s * CH_T + h * CH_Q, CH_Q)], dstv)
        lax.fori_loop(0, CH_Q // 4, _quad, 0)

    for i in range(4):
        pltpu.make_async_copy(y0_hbm.at[pl.ds(0, CHUNK)], bufs[i],
                              ssems[i]).wait()

    plsc.subcore_barrier()

    pltpu.sync_copy(coef_hbm, coefv)
    idx16 = jnp.zeros((16,), _i32)
    c0v = plsc.load_gather(coefv, [idx16])
    c1v = plsc.load_gather(coefv, [idx16 + 1])
    c2v = plsc.load_gather(coefv, [idx16 + 2])

    def _comb_chunk(y_hbm, x_hbm, base, lbase, nrows):
        pltpu.sync_copy(agg.at[pl.ds(base, nrows)], rows_a.at[pl.ds(0, nrows)])
        pltpu.sync_copy(y_hbm.at[pl.ds(base, nrows)], rows_b.at[pl.ds(0, nrows)])
        pltpu.sync_copy(x_hbm.at[pl.ds(base, nrows)], rows_c.at[pl.ds(0, nrows)])

        def _r(r, carry):
            dv = plsc.load_gather(dinv_lv, [jnp.full((16,), lbase + r, _i32)])
            t = c1v * dv
            for cb in range(DH // 16):
                sl = pl.ds(cb * 16, 16)
                rows_d[r, sl] = (c0v * rows_b[r, sl] + t * rows_a[r, sl]
                                 + c2v * rows_c[r, sl])
            return carry

        lax.fori_loop(0, nrows, _r, 0)
        pltpu.sync_copy(rows_d.at[pl.ds(0, nrows)],
                        out_hbm.at[pl.ds(base, nrows), pl.ds(c * DH, DH)])

    def _comb_half(y_hbm, x_hbm, nfull, tail):
        def _q(q, carry):
            _comb_chunk(y_hbm, x_hbm, s * SHARE + q * CHUNK, q * CHUNK, CHUNK)
            return carry

        lax.fori_loop(0, nfull, _q, 0)
        if tail:
            _comb_chunk(y_hbm, x_hbm, s * SHARE + nfull * CHUNK,
                        nfull * CHUNK, tail)

    @pl.when(jnp.logical_and(c == 0, s < NS - 1))
    def _():
        _comb_half(y0_hbm, x0_hbm, SHARE // CHUNK, 0)

    @pl.when(jnp.logical_and(c == 0, s == NS - 1))
    def _():
        _comb_half(y0_hbm, x0_hbm, 6, 16)

    @pl.when(jnp.logical_and(c == 1, s < NS - 1))
    def _():
        _comb_half(y1_hbm, x1_hbm, SHARE // CHUNK, 0)

    @pl.when(jnp.logical_and(c == 1, s == NS - 1))
    def _():
        _comb_half(y1_hbm, x1_hbm, 6, 16)


_fused = functools.partial(
    pl.kernel,
    out_type=jax.ShapeDtypeStruct((N_NODES, D), _f32),
    mesh=_mesh,
    scratch_types=[
        pltpu.VMEM((CH_Q, CHUNK), _i32),
        pltpu.VMEM((CH_Q, CHUNK), _i32),
        pltpu.VMEM((HB,), _f32),
        pltpu.VMEM((HB_ROWS // 4, D), _f32),
        pltpu.VMEM((4, 20), _i32),
        pltpu.VMEM((5, D), _f32),
        pltpu.VMEM((5, D), _f32),
        pltpu.VMEM((SHARE,), _f32),
        pltpu.VMEM((CHUNK,), _i32),
        pltpu.VMEM((16,), _f32),
        pltpu.VMEM((CHUNK, DH), _f32),
        pltpu.VMEM((CHUNK, DH), _f32),
        pltpu.VMEM((CHUNK, DH), _f32),
        pltpu.VMEM((CHUNK, DH), _f32),
        pltpu.SemaphoreType.DMA,
        pltpu.SemaphoreType.DMA,
        pltpu.SemaphoreType.DMA,
        pltpu.SemaphoreType.DMA,
        pltpu.SemaphoreType.DMA,
        pltpu.SemaphoreType.DMA,
        pltpu.SemaphoreType.DMA,
        pltpu.SemaphoreType.DMA,
        pltpu.VMEM_SHARED((HB_ROWS, D), _f32),
        pltpu.VMEM_SHARED((AGG_ROWS, DH), _f32),
        pltpu.VMEM_SHARED((AGG_ROWS, DH), _f32),
    ],
    compiler_params=pltpu.CompilerParams(needs_layout_passes=False,
                                         use_tc_tiling_on_sc=False),
)(_fused_body)


def kernel(edge_index, Y, X, alp, lam):
    src = edge_index[0]
    dst = edge_index[1]

    npad = E_PAD - N_EDGES
    ar = jnp.arange(npad, dtype=_i32)
    src_p = jnp.concatenate([src, ar % N_NODES]).reshape(N_CHUNK_ROWS, CHUNK)
    dst_p = jnp.concatenate([dst, N_NODES + ar % (AGG_ROWS - N_NODES)]
                            ).reshape(N_CHUNK_ROWS, CHUNK)

    zpad = jnp.zeros((AGG_ROWS - N_NODES, DH), _f32)
    y0 = jnp.concatenate([Y[:, :DH], zpad])
    y1 = jnp.concatenate([Y[:, DH:], zpad])

    al = alp * lam
    coef = jnp.concatenate([
        jnp.stack([1.0 - al - alp, al, alp]).astype(_f32),
        jnp.zeros((13,), _f32),
    ])

    return _fused(dst_p, src_p, y0, y1, X[:, :DH], X[:, DH:], coef)


# speedup vs baseline: 1.0211x; 1.0211x over previous
import functools

import jax
import jax.numpy as jnp
from jax import lax
from jax.experimental import pallas as pl
from jax.experimental.pallas import tpu as pltpu
from jax.experimental.pallas import tpu_sc as plsc

N_NODES = 10000
N_EDGES = 320000
D = 128

NC = 2
NS = 16
NW = NC * NS

CHUNK = 80
E_PAD = 327680
N_CHUNK_ROWS = E_PAD // CHUNK
CH_T = N_CHUNK_ROWS // NS
CH_Q = CH_T // 4

HB = 10240
HB_ROWS = HB // D
AGG_ROWS = 10240
SHARE = AGG_ROWS // NS
DH = 64

RB = 400
GRID = N_NODES // RB

_mesh = plsc.VectorSubcoreMesh(core_axis_name="c", subcore_axis_name="s")

_f32 = jnp.float32
_i32 = jnp.int32


def _fused_body(dst2d_hbm, src2d_hbm, y0_hbm, y1_hbm,
                aggh_hbm, dinv_hbm,
                dstv, sidx, hist1d, packb, rowids, histv, dinvv, dinv_lv,
                dumidx,
                rows_a, rows_b, rows_c, rows_d, sem_a, sem_b, sem_c, sem_d,
                sem_e, sem_f, sem_g, sem_h, hist_sh, ys_sp, agg):
    c = lax.axis_index("c")
    s = lax.axis_index("s")

    zeros16 = jnp.zeros((16,), _f32)
    ones16 = jnp.ones((16,), _f32)

    def _z1(i, carry):
        hist1d[pl.ds(i * 16, 16)] = zeros16
        return carry

    lax.fori_loop(0, HB // 16, _z1, 0)

    def _zp(r, carry):
        for cb in range(8):
            packb[r, pl.ds(cb * 16, 16)] = zeros16
        return carry

    lax.fori_loop(0, HB_ROWS // 4, _zp, 0)

    def _za(r, carry):
        for cb in range(DH // 16):
            rows_a[r, pl.ds(cb * 16, 16)] = zeros16
        return carry

    lax.fori_loop(0, CHUNK, _za, 0)

    for p in range(4):
        rowids[p, pl.ds(0, 16)] = lax.iota(_i32, 16) + p * 20
        rowids[p, pl.ds(4, 16)] = lax.iota(_i32, 16) + (p * 20 + 4)

    for k in range(SHARE // CHUNK):
        pltpu.sync_copy(rows_a, agg.at[pl.ds(s * SHARE + k * CHUNK, CHUNK)])

    @pl.when(s == 0)
    def _():
        for p in range(4):
            pltpu.sync_copy(packb, hist_sh.at[pl.ds(p * 20, 20)])

    plsc.subcore_barrier()

    def _hrow(r, carry):
        for cb in range(CHUNK // 16):
            idx = dstv[r, pl.ds(cb * 16, 16)]
            plsc.addupdate_scatter(hist1d, [idx], ones16)
        return carry

    for h in range(4):
        pltpu.sync_copy(dst2d_hbm.at[pl.ds(s * CH_T + h * CH_Q, CH_Q)], dstv)
        lax.fori_loop(0, CH_Q, _hrow, 0)

    for p in range(4):
        def _pk(r, carry, p=p):
            for cb in range(8):
                packb[r, pl.ds(cb * 16, 16)] = \
                    hist1d[pl.ds((p * 20 + r) * D + cb * 16, 16)]
            return carry

        lax.fori_loop(0, HB_ROWS // 4, _pk, 0)
        pltpu.sync_copy(packb, hist_sh.at[rowids.at[p]], add=True)

    plsc.subcore_barrier()

    pltpu.sync_copy(hist_sh.at[pl.ds(s * 5, 5)], histv)
    half16 = jnp.full((16,), 0.5, _f32)
    t32 = jnp.full((16,), 1.5, _f32)
    magic = jnp.full((16,), 0x5F3759DF, _i32)
    for r in range(5):
        for cb in range(8):
            x = histv[r, pl.ds(cb * 16, 16)]
            yv = plsc.bitcast(magic - (plsc.bitcast(x, _i32) >> 1), _f32)
            for _ in range(3):
                yv = yv * (t32 - half16 * x * yv * yv)
            dv = jnp.where(x > 0, yv, 0.0)
            dinvv[r, pl.ds(cb * 16, 16)] = dv
            dinv_lv[pl.ds((r * 8 + cb) * 16, 16)] = dv

    @pl.when(c == 0)
    def _():
        pltpu.sync_copy(dinvv, dinv_hbm.at[pl.ds(s * 5, 5)])

    def _scale(y_hbm):
        def _q(q, carry):
            base = s * SHARE + q * CHUNK

            pltpu.sync_copy(y_hbm.at[pl.ds(base, CHUNK)], rows_a)

            def _r(r, carry2):
                gi = jnp.full((16,), q * CHUNK + r, _i32)
                dv = plsc.load_gather(dinv_lv, [gi])
                for cb in range(DH // 16):
                    rows_a[r, pl.ds(cb * 16, 16)] = rows_a[r, pl.ds(cb * 16, 16)] * dv
                return carry2

            lax.fori_loop(0, CHUNK, _r, 0)
            pltpu.sync_copy(rows_a, ys_sp.at[pl.ds(base, CHUNK)])
            return carry

        lax.fori_loop(0, SHARE // CHUNK, _q, 0)

    @pl.when(c == 0)
    def _():
        _scale(y0_hbm)

    @pl.when(c == 1)
    def _():
        _scale(y1_hbm)

    plsc.subcore_barrier()

    bufs = (rows_a, rows_b, rows_c, rows_d)
    gsems = (sem_a, sem_b, sem_c, sem_d)
    ssems = (sem_e, sem_f, sem_g, sem_h)

    for g in range(CHUNK // 16):
        dumidx[pl.ds(g * 16, 16)] = lax.iota(_i32, 16) + (N_NODES + g * 16)
    for i in range(4):
        pltpu.async_copy(bufs[i], agg.at[dumidx], ssems[i], add=True)

    def _quad(k, carry):
        j0 = 4 * k
        for i in range(4):
            pltpu.make_async_copy(y0_hbm.at[pl.ds(0, CHUNK)], bufs[i],
                                  ssems[i]).wait()
        gs = [pltpu.async_copy(ys_sp.at[sidx.at[j0 + i]], bufs[i], gsems[i])
              for i in range(4)]
        for i in range(4):
            gs[i].wait()
            pltpu.async_copy(bufs[i], agg.at[dstv.at[j0 + i]], ssems[i],
                             add=True)
        return carry

    for h in range(4):
        pltpu.sync_copy(src2d_hbm.at[pl.ds(s * CH_T + h * CH_Q, CH_Q)], sidx)
        pltpu.sync_copy(dst2d_hbm.at[pl.ds(s * CH_T + h * CH_Q, CH_Q)], dstv)
        lax.fori_loop(0, CH_Q // 4, _quad, 0)

    for i in range(4):
        pltpu.make_async_copy(y0_hbm.at[pl.ds(0, CHUNK)], bufs[i],
                              ssems[i]).wait()

    plsc.subcore_barrier()

    pltpu.sync_copy(agg.at[pl.ds(s * SHARE, SHARE)],
                    aggh_hbm.at[c, pl.ds(s * SHARE, SHARE)])


_fused = functools.partial(
    pl.kernel,
    out_type=(
        jax.ShapeDtypeStruct((NC, AGG_ROWS, DH), _f32),
        jax.ShapeDtypeStruct((HB_ROWS, D), _f32),
    ),
    mesh=_mesh,
    scratch_types=[
        pltpu.VMEM((CH_Q, CHUNK), _i32),
        pltpu.VMEM((CH_Q, CHUNK), _i32),
        pltpu.VMEM((HB,), _f32),
        pltpu.VMEM((HB_ROWS // 4, D), _f32),
        pltpu.VMEM((4, 20), _i32),
        pltpu.VMEM((5, D), _f32),
        pltpu.VMEM((5, D), _f32),
        pltpu.VMEM((SHARE,), _f32),
        pltpu.VMEM((CHUNK,), _i32),
        pltpu.VMEM((CHUNK, DH), _f32),
        pltpu.VMEM((CHUNK, DH), _f32),
        pltpu.VMEM((CHUNK, DH), _f32),
        pltpu.VMEM((CHUNK, DH), _f32),
        pltpu.SemaphoreType.DMA,
        pltpu.SemaphoreType.DMA,
        pltpu.SemaphoreType.DMA,
        pltpu.SemaphoreType.DMA,
        pltpu.SemaphoreType.DMA,
        pltpu.SemaphoreType.DMA,
        pltpu.SemaphoreType.DMA,
        pltpu.SemaphoreType.DMA,
        pltpu.VMEM_SHARED((HB_ROWS, D), _f32),
        pltpu.VMEM_SHARED((AGG_ROWS, DH), _f32),
        pltpu.VMEM_SHARED((AGG_ROWS, DH), _f32),
    ],
    compiler_params=pltpu.CompilerParams(needs_layout_passes=False,
                                         use_tc_tiling_on_sc=False),
)(_fused_body)


def _comb_body(coef_ref, y_ref, x_ref, dinv_ref, ap_ref, out_ref):
    c0, c1, c2 = coef_ref[0], coef_ref[1], coef_ref[2]
    dinv = dinv_ref[...]
    out_ref[:, :DH] = (c0 * y_ref[:, :DH]
                       + c1 * (dinv * ap_ref[0])
                       + c2 * x_ref[:, :DH])
    out_ref[:, DH:] = (c0 * y_ref[:, DH:]
                       + c1 * (dinv * ap_ref[1])
                       + c2 * x_ref[:, DH:])


_comb = pl.pallas_call(
    _comb_body,
    grid=(GRID,),
    in_specs=[
        pl.BlockSpec(memory_space=pltpu.MemorySpace.SMEM),
        pl.BlockSpec((RB, D), lambda i: (i, 0)),
        pl.BlockSpec((RB, D), lambda i: (i, 0)),
        pl.BlockSpec((RB, 1), lambda i: (i, 0)),
        pl.BlockSpec((2, RB, DH), lambda i: (0, i, 0)),
    ],
    out_specs=pl.BlockSpec((RB, D), lambda i: (i, 0)),
    out_shape=jax.ShapeDtypeStruct((N_NODES, D), _f32),
)


def kernel(edge_index, Y, X, alp, lam):
    src = edge_index[0]
    dst = edge_index[1]

    npad = E_PAD - N_EDGES
    ar = jnp.arange(npad, dtype=_i32)
    src_p = jnp.concatenate([src, ar % N_NODES]).reshape(N_CHUNK_ROWS, CHUNK)
    dst_p = jnp.concatenate([dst, N_NODES + ar % (AGG_ROWS - N_NODES)]
                            ).reshape(N_CHUNK_ROWS, CHUNK)

    zpad = jnp.zeros((AGG_ROWS - N_NODES, DH), _f32)
    y0 = jnp.concatenate([Y[:, :DH], zpad])
    y1 = jnp.concatenate([Y[:, DH:], zpad])

    aggh, dinv2d = _fused(dst_p, src_p, y0, y1)
    dinv_col = dinv2d.reshape(HB)[:N_NODES].reshape(N_NODES, 1)

    al = alp * lam
    coef = jnp.stack([1.0 - al - alp, al, alp]).astype(_f32)
    return _comb(coef, Y, X, dinv_col, aggh)


# speedup vs baseline: 1.0536x; 1.0318x over previous
import functools

import jax
import jax.numpy as jnp
from jax import lax
from jax.experimental import pallas as pl
from jax.experimental.pallas import tpu as pltpu
from jax.experimental.pallas import tpu_sc as plsc

N_NODES = 10000
N_EDGES = 320000
D = 128

NC = 2
NS = 16
NW = NC * NS

CHUNK = 80
E_PAD = 327680
N_CHUNK_ROWS = E_PAD // CHUNK
CH_T = N_CHUNK_ROWS // NS
CH_Q = CH_T // 4

HB = 10240
HB_ROWS = HB // D
AGG_ROWS = 10240
SHARE = AGG_ROWS // NS
DH = 64

RB = 400
GRID = N_NODES // RB

_mesh = plsc.VectorSubcoreMesh(core_axis_name="c", subcore_axis_name="s")

_f32 = jnp.float32
_i32 = jnp.int32


def _fused_body(dst2d_hbm, src2d_hbm, y0_hbm, y1_hbm,
                aggh_hbm, dinv_hbm,
                dstv, sidx, hist1d, packb, rowids, histv, dinvv, dinv_lv,
                dumidx,
                rows_a, rows_b, rows_c, rows_d, sem_a, sem_b, sem_c, sem_d,
                sem_e, sem_f, sem_g, sem_h, hist_sh, ys_sp, agg):
    c = lax.axis_index("c")
    s = lax.axis_index("s")

    zeros16 = jnp.zeros((16,), _f32)
    ones16 = jnp.ones((16,), _f32)

    def _z1(i, carry):
        hist1d[pl.ds(i * 16, 16)] = zeros16
        return carry

    lax.fori_loop(0, HB // 16, _z1, 0)

    def _zp(r, carry):
        for cb in range(8):
            packb[r, pl.ds(cb * 16, 16)] = zeros16
        return carry

    lax.fori_loop(0, HB_ROWS // 4, _zp, 0)

    def _za(r, carry):
        for cb in range(DH // 16):
            rows_a[r, pl.ds(cb * 16, 16)] = zeros16
        return carry

    lax.fori_loop(0, CHUNK, _za, 0)

    for p in range(4):
        rowids[p, pl.ds(0, 16)] = lax.iota(_i32, 16) + p * 20
        rowids[p, pl.ds(4, 16)] = lax.iota(_i32, 16) + (p * 20 + 4)

    for k in range(SHARE // CHUNK):
        pltpu.sync_copy(rows_a, agg.at[pl.ds(s * SHARE + k * CHUNK, CHUNK)])

    @pl.when(s == 0)
    def _():
        for p in range(4):
            pltpu.sync_copy(packb, hist_sh.at[pl.ds(p * 20, 20)])

    plsc.subcore_barrier()

    def _make_hrow(buf):
        def _hrow(r, carry):
            for cb in range(CHUNK // 16):
                idx = buf[r, pl.ds(cb * 16, 16)]
                plsc.addupdate_scatter(hist1d, [idx], ones16)
            return carry
        return _hrow

    hbufs = (dstv, sidx)
    hcp = [None] * 4
    hcp[0] = pltpu.async_copy(dst2d_hbm.at[pl.ds(s * CH_T, CH_Q)], dstv, sem_a)
    for h in range(4):
        if h + 1 < 4:
            hcp[h + 1] = pltpu.async_copy(
                dst2d_hbm.at[pl.ds(s * CH_T + (h + 1) * CH_Q, CH_Q)],
                hbufs[(h + 1) % 2], sem_b if (h + 1) % 2 else sem_a)
        hcp[h].wait()
        lax.fori_loop(0, CH_Q, _make_hrow(hbufs[h % 2]), 0)

    for p in range(4):
        def _pk(r, carry, p=p):
            for cb in range(8):
                packb[r, pl.ds(cb * 16, 16)] = \
                    hist1d[pl.ds((p * 20 + r) * D + cb * 16, 16)]
            return carry

        lax.fori_loop(0, HB_ROWS // 4, _pk, 0)
        pltpu.sync_copy(packb, hist_sh.at[rowids.at[p]], add=True)

    plsc.subcore_barrier()

    pltpu.sync_copy(hist_sh.at[pl.ds(s * 5, 5)], histv)
    half16 = jnp.full((16,), 0.5, _f32)
    t32 = jnp.full((16,), 1.5, _f32)
    magic = jnp.full((16,), 0x5F3759DF, _i32)
    for r in range(5):
        for cb in range(8):
            x = histv[r, pl.ds(cb * 16, 16)]
            yv = plsc.bitcast(magic - (plsc.bitcast(x, _i32) >> 1), _f32)
            for _ in range(3):
                yv = yv * (t32 - half16 * x * yv * yv)
            dv = jnp.where(x > 0, yv, 0.0)
            dinvv[r, pl.ds(cb * 16, 16)] = dv
            dinv_lv[pl.ds((r * 8 + cb) * 16, 16)] = dv

    @pl.when(c == 0)
    def _():
        pltpu.sync_copy(dinvv, dinv_hbm.at[pl.ds(s * 5, 5)])

    def _scale(y_hbm):
        sbufs = (rows_a, rows_b)
        ssem = (sem_a, sem_b)
        nq = SHARE // CHUNK
        cps = [None] * nq
        cps[0] = pltpu.async_copy(y_hbm.at[pl.ds(s * SHARE, CHUNK)],
                                  rows_a, sem_a)
        for q in range(nq):
            buf = sbufs[q % 2]
            if q + 1 < nq:
                cps[q + 1] = pltpu.async_copy(
                    y_hbm.at[pl.ds(s * SHARE + (q + 1) * CHUNK, CHUNK)],
                    sbufs[(q + 1) % 2], ssem[(q + 1) % 2])
            cps[q].wait()

            def _r(r, carry2, q=q, buf=buf):
                gi = jnp.full((16,), q * CHUNK + r, _i32)
                dv = plsc.load_gather(dinv_lv, [gi])
                for cb in range(DH // 16):
                    buf[r, pl.ds(cb * 16, 16)] = buf[r, pl.ds(cb * 16, 16)] * dv
                return carry2

            lax.fori_loop(0, CHUNK, _r, 0)
            pltpu.sync_copy(buf, ys_sp.at[pl.ds(s * SHARE + q * CHUNK, CHUNK)])

    @pl.when(c == 0)
    def _():
        _scale(y0_hbm)

    @pl.when(c == 1)
    def _():
        _scale(y1_hbm)

    plsc.subcore_barrier()

    bufs = (rows_a, rows_b, rows_c, rows_d)
    gsems = (sem_a, sem_b, sem_c, sem_d)
    ssems = (sem_e, sem_f, sem_g, sem_h)

    for g in range(CHUNK // 16):
        dumidx[pl.ds(g * 16, 16)] = lax.iota(_i32, 16) + (N_NODES + g * 16)
    for i in range(4):
        pltpu.async_copy(bufs[i], agg.at[dumidx], ssems[i], add=True)

    def _quad(k, carry):
        j0 = 4 * k
        for i in range(4):
            pltpu.make_async_copy(y0_hbm.at[pl.ds(0, CHUNK)], bufs[i],
                                  ssems[i]).wait()
        gs = [pltpu.async_copy(ys_sp.at[sidx.at[j0 + i]], bufs[i], gsems[i])
              for i in range(4)]
        for i in range(4):
            gs[i].wait()
            pltpu.async_copy(bufs[i], agg.at[dstv.at[j0 + i]], ssems[i],
                             add=True)
        return carry

    for h in range(4):
        pltpu.sync_copy(src2d_hbm.at[pl.ds(s * CH_T + h * CH_Q, CH_Q)], sidx)
        pltpu.sync_copy(dst2d_hbm.at[pl.ds(s * CH_T + h * CH_Q, CH_Q)], dstv)
        lax.fori_loop(0, CH_Q // 4, _quad, 0)

    for i in range(4):
        pltpu.make_async_copy(y0_hbm.at[pl.ds(0, CHUNK)], bufs[i],
                              ssems[i]).wait()

    plsc.subcore_barrier()

    pltpu.sync_copy(agg.at[pl.ds(s * SHARE, SHARE)],
                    aggh_hbm.at[c, pl.ds(s * SHARE, SHARE)])


_fused = functools.partial(
    pl.kernel,
    out_type=(
        jax.ShapeDtypeStruct((NC, AGG_ROWS, DH), _f32),
        jax.ShapeDtypeStruct((HB_ROWS, D), _f32),
    ),
    mesh=_mesh,
    scratch_types=[
        pltpu.VMEM((CH_Q, CHUNK), _i32),
        pltpu.VMEM((CH_Q, CHUNK), _i32),
        pltpu.VMEM((HB,), _f32),
        pltpu.VMEM((HB_ROWS // 4, D), _f32),
        pltpu.VMEM((4, 20), _i32),
        pltpu.VMEM((5, D), _f32),
        pltpu.VMEM((5, D), _f32),
        pltpu.VMEM((SHARE,), _f32),
        pltpu.VMEM((CHUNK,), _i32),
        pltpu.VMEM((CHUNK, DH), _f32),
        pltpu.VMEM((CHUNK, DH), _f32),
        pltpu.VMEM((CHUNK, DH), _f32),
        pltpu.VMEM((CHUNK, DH), _f32),
        pltpu.SemaphoreType.DMA,
        pltpu.SemaphoreType.DMA,
        pltpu.SemaphoreType.DMA,
        pltpu.SemaphoreType.DMA,
        pltpu.SemaphoreType.DMA,
        pltpu.SemaphoreType.DMA,
        pltpu.SemaphoreType.DMA,
        pltpu.SemaphoreType.DMA,
        pltpu.VMEM_SHARED((HB_ROWS, D), _f32),
        pltpu.VMEM_SHARED((AGG_ROWS, DH), _f32),
        pltpu.VMEM_SHARED((AGG_ROWS, DH), _f32),
    ],
    compiler_params=pltpu.CompilerParams(needs_layout_passes=False,
                                         use_tc_tiling_on_sc=False),
)(_fused_body)


def _comb_body(coef_ref, y_ref, x_ref, dinv_ref, ap_ref, out_ref):
    c0, c1, c2 = coef_ref[0], coef_ref[1], coef_ref[2]
    dinv = dinv_ref[...]
    out_ref[:, :DH] = (c0 * y_ref[:, :DH]
                       + c1 * (dinv * ap_ref[0])
                       + c2 * x_ref[:, :DH])
    out_ref[:, DH:] = (c0 * y_ref[:, DH:]
                       + c1 * (dinv * ap_ref[1])
                       + c2 * x_ref[:, DH:])


_comb = pl.pallas_call(
    _comb_body,
    grid=(GRID,),
    in_specs=[
        pl.BlockSpec(memory_space=pltpu.MemorySpace.SMEM),
        pl.BlockSpec((RB, D), lambda i: (i, 0)),
        pl.BlockSpec((RB, D), lambda i: (i, 0)),
        pl.BlockSpec((RB, 1), lambda i: (i, 0)),
        pl.BlockSpec((2, RB, DH), lambda i: (0, i, 0)),
    ],
    out_specs=pl.BlockSpec((RB, D), lambda i: (i, 0)),
    out_shape=jax.ShapeDtypeStruct((N_NODES, D), _f32),
)


def kernel(edge_index, Y, X, alp, lam):
    src = edge_index[0]
    dst = edge_index[1]

    npad = E_PAD - N_EDGES
    ar = jnp.arange(npad, dtype=_i32)
    src_p = jnp.concatenate([src, ar % N_NODES]).reshape(N_CHUNK_ROWS, CHUNK)
    dst_p = jnp.concatenate([dst, N_NODES + ar % (AGG_ROWS - N_NODES)]
                            ).reshape(N_CHUNK_ROWS, CHUNK)

    zpad = jnp.zeros((AGG_ROWS - N_NODES, DH), _f32)
    y0 = jnp.concatenate([Y[:, :DH], zpad])
    y1 = jnp.concatenate([Y[:, DH:], zpad])

    aggh, dinv2d = _fused(dst_p, src_p, y0, y1)
    dinv_col = dinv2d.reshape(HB)[:N_NODES].reshape(N_NODES, 1)

    al = alp * lam
    coef = jnp.stack([1.0 - al - alp, al, alp]).astype(_f32)
    return _comb(coef, Y, X, dinv_col, aggh)
